# baseline (speedup 1.0000x reference)
import functools

import jax
import jax.numpy as jnp
from jax import lax
from jax.experimental import pallas as pl
from jax.experimental.pallas import tpu as pltpu
from jax.experimental.pallas import tpu_sc as plsc

N_NODES = 1_000_000
N_STEPS_BUF = 1000
LANES = 16

TC_BLOCK = 131_072
TC_GRID = -(-N_NODES // TC_BLOCK)
LAST_VALID = N_NODES - (TC_GRID - 1) * TC_BLOCK


def _fused_body(f_ref, m_ref, s_ref, t_ref, o_ref, part_ref, acc_ref):
    i = pl.program_id(0)
    o_ref[...] = jnp.where(s_ref[...] == 0, f_ref[...], m_ref[...])

    @pl.when(i == 0)
    def _init():
        acc_ref[0] = 0.0

    @pl.when(i < TC_GRID - 1)
    def _full():
        acc_ref[0] += jnp.sum(f_ref[...] * t_ref[...].astype(jnp.float32))

    @pl.when(i == TC_GRID - 1)
    def _last():
        sl = pl.ds(0, LAST_VALID)
        acc_ref[0] += jnp.sum(f_ref[sl] * t_ref[sl].astype(jnp.float32))
        part_ref[...] = jnp.full((LANES,), acc_ref[0], jnp.float32)


def _tc_fused(fringe, minf, status, term):
    spec = pl.BlockSpec((TC_BLOCK,), lambda i: (i,))
    return pl.pallas_call(
        _fused_body,
        grid=(TC_GRID,),
        in_specs=[spec, spec, spec, spec],
        out_specs=[
            spec,
            pl.BlockSpec((LANES,), lambda i: (0,)),
        ],
        out_shape=[
            jax.ShapeDtypeStruct((N_NODES,), jnp.float32),
            jax.ShapeDtypeStruct((LANES,), jnp.float32),
        ],
        scratch_shapes=[pltpu.SMEM((1,), jnp.float32)],
    )(fringe, minf, status, term)


def _sc_scatter_body(flux_hbm, part_hbm, step_hbm, out_hbm,
                     flux_v, part_v, step_v):
    wid = lax.axis_index("s") * 2 + lax.axis_index("c")

    @pl.when(wid == 0)
    def _go():
        pltpu.sync_copy(flux_hbm, flux_v.at[pl.ds(0, N_STEPS_BUF)])
        pltpu.sync_copy(part_hbm, part_v)
        pltpu.sync_copy(step_hbm, step_v)
        step_vec = step_v[...]
        total_vec = part_v[...]
        lane = lax.iota(jnp.int32, LANES)

        def overwrite(j, _):
            sl = pl.ds(j * LANES, LANES)
            gidx = lane + j * LANES
            flux_v[sl] = jnp.where(gidx == step_vec, total_vec, flux_v[sl])
            return 0

        lax.fori_loop(0, 63, overwrite, 0)
        pltpu.sync_copy(flux_v.at[pl.ds(0, N_STEPS_BUF)], out_hbm)


@functools.partial(
    pl.kernel,
    out_type=jax.ShapeDtypeStruct((N_STEPS_BUF,), jnp.float32),
    mesh=plsc.VectorSubcoreMesh(core_axis_name="c", subcore_axis_name="s"),
    scratch_types=[
        pltpu.VMEM((1024,), jnp.float32),
        pltpu.VMEM((LANES,), jnp.float32),
        pltpu.VMEM((LANES,), jnp.int32),
    ],
)
def _sc_scatter(*args):
    _sc_scatter_body(*args)


def kernel(fringe_thickness, min_fringe_thickness, fluxes, node_is_terminus,
           status_at_node, current_step):
    cleared, partial = _tc_fused(fringe_thickness, min_fringe_thickness,
                                 status_at_node, node_is_terminus)
    step_vec = jnp.full((LANES,), current_step, jnp.int32)
    updated = _sc_scatter(fluxes, partial, step_vec)
    return cleared, updated


# speedup vs baseline: 1.1157x; 1.1157x over previous
import functools

import jax
import jax.numpy as jnp
from jax import lax
from jax.experimental import pallas as pl
from jax.experimental.pallas import tpu as pltpu
from jax.experimental.pallas import tpu_sc as plsc

N_NODES = 1_000_000
LANES = 16

CHUNK = 8_192
N_WORKERS = 32
ITERS = 3
SC_CHUNKS = N_WORKERS * ITERS
SC_NODES = SC_CHUNKS * CHUNK
GROUPS = 4

TC_BLOCK = 262_144
TC_GRID = -(-N_NODES // TC_BLOCK)
TAIL_BLOCK = SC_NODES // TC_BLOCK
TAIL_VALID = N_NODES - SC_NODES


def _sc_reduce_body(fringe_hbm, term_hbm, part_hbm,
                    f0, f1, t0, t1, acc_v, isem0, isem1):
    wid = lax.axis_index("s") * 2 + lax.axis_index("c")
    f_v, t_v = (f0, f1), (t0, t1)
    isems = (isem0, isem1)

    def in_copies(slot, chunk):
        off = chunk * CHUNK
        sl = pl.ds(off, CHUNK)
        return (
            pltpu.make_async_copy(fringe_hbm.at[sl], f_v[slot], isems[slot]),
            pltpu.make_async_copy(term_hbm.at[sl], t_v[slot], isems[slot]),
        )

    for c in in_copies(0, wid):
        c.start()

    accs = (jnp.zeros((LANES,), jnp.float32),) * GROUPS
    for it in range(ITERS):
        chunk = wid + it * N_WORKERS
        slot = it % 2

        if it + 1 < ITERS:
            for c in in_copies(1 - slot, chunk + N_WORKERS):
                c.start()

        for c in in_copies(slot, chunk):
            c.wait()

        @plsc.parallel_loop(0, CHUNK, step=GROUPS * LANES, unroll=2,
                            carry=accs)
        def body(j, carry):
            new = []
            for g in range(GROUPS):
                sl = pl.ds(j + g * LANES, LANES)
                new.append(carry[g]
                           + f_v[slot][sl] * t_v[slot][sl].astype(jnp.float32))
            return tuple(new)

        accs = body

    a0, a1, a2, a3 = accs
    acc_v[...] = (a0 + a1) + (a2 + a3)
    pltpu.sync_copy(acc_v, part_hbm.at[wid])


@functools.partial(
    pl.kernel,
    out_type=jax.ShapeDtypeStruct((N_WORKERS, LANES), jnp.float32),
    mesh=plsc.VectorSubcoreMesh(core_axis_name="c", subcore_axis_name="s"),
    scratch_types=[
        pltpu.VMEM((CHUNK,), jnp.float32),
        pltpu.VMEM((CHUNK,), jnp.float32),
        pltpu.VMEM((CHUNK,), jnp.int32),
        pltpu.VMEM((CHUNK,), jnp.int32),
        pltpu.VMEM((LANES,), jnp.float32),
        pltpu.SemaphoreType.DMA,
        pltpu.SemaphoreType.DMA,
    ],
)
def _sc_reduce(*args):
    _sc_reduce_body(*args)


def _select_body(f_ref, m_ref, s_ref, o_ref):
    o_ref[...] = jnp.where(s_ref[...] == 0, f_ref[...], m_ref[...])


def _tc_select(fringe, minf, status):
    spec = pl.BlockSpec((TC_BLOCK,), lambda i: (i,))
    return pl.pallas_call(
        _select_body,
        grid=(TC_GRID,),
        in_specs=[spec, spec, spec],
        out_specs=spec,
        out_shape=jax.ShapeDtypeStruct((N_NODES,), jnp.float32),
    )(fringe, minf, status)


def _flux_body(step_ref, f_ref, t_ref, part_ref, flux_ref, out_ref):
    sl = pl.ds(0, TAIL_VALID)
    tail_sum = jnp.sum(f_ref[sl] * t_ref[sl].astype(jnp.float32))
    total = tail_sum + jnp.sum(part_ref[...])
    step = step_ref[0, 0]
    cols = lax.broadcasted_iota(jnp.int32, (1, 1000), 1)
    out_ref[...] = jnp.where(cols == step, total, flux_ref[...])


def _flux_update(step2d, fringe, term, partials, flux2d):
    tail_spec = pl.BlockSpec((TC_BLOCK,), lambda i: (TAIL_BLOCK,))
    return pl.pallas_call(
        _flux_body,
        grid=(1,),
        out_shape=jax.ShapeDtypeStruct((1, 1000), jnp.float32),
        in_specs=[
            pl.BlockSpec(memory_space=pltpu.SMEM),
            tail_spec,
            tail_spec,
            pl.BlockSpec((N_WORKERS, LANES), lambda i: (0, 0)),
            pl.BlockSpec((1, 1000), lambda i: (0, 0)),
        ],
        out_specs=pl.BlockSpec((1, 1000), lambda i: (0, 0)),
    )(step2d, fringe, term, partials, flux2d)


def kernel(fringe_thickness, min_fringe_thickness, fluxes, node_is_terminus,
           status_at_node, current_step):
    partials = _sc_reduce(fringe_thickness, node_is_terminus)
    cleared = _tc_select(fringe_thickness, min_fringe_thickness, status_at_node)
    step2d = jnp.asarray(current_step, jnp.int32).reshape(1, 1)
    flux2d = fluxes.reshape(1, 1000)
    out2d = _flux_update(step2d, fringe_thickness, node_is_terminus,
                         partials, flux2d)
    return cleared, out2d.reshape(fluxes.shape)


# speedup vs baseline: 1.1792x; 1.0569x over previous
import functools

import jax
import jax.numpy as jnp
from jax import lax
from jax.experimental import pallas as pl
from jax.experimental.pallas import tpu as pltpu
from jax.experimental.pallas import tpu_sc as plsc

N_NODES = 1_000_000
LANES = 16

CHUNK = 7_808
N_WORKERS = 32
ITERS = 4
SC_CHUNKS = N_WORKERS * ITERS
SC_NODES = SC_CHUNKS * CHUNK
GROUPS = 4

TC_BLOCK = 262_144
TC_GRID = -(-N_NODES // TC_BLOCK)
TAIL_CHUNK = 1024
TAIL_BLOCK = SC_NODES // TAIL_CHUNK
TAIL_VALID = N_NODES - SC_NODES


def _sc_reduce_body(fringe_hbm, term_hbm, part_hbm,
                    f0, f1, t0, t1, acc_v, isem0, isem1):
    wid = lax.axis_index("s") * 2 + lax.axis_index("c")
    f_v, t_v = (f0, f1), (t0, t1)
    isems = (isem0, isem1)

    def in_copies(slot, chunk):
        off = chunk * CHUNK
        sl = pl.ds(off, CHUNK)
        return (
            pltpu.make_async_copy(fringe_hbm.at[sl], f_v[slot], isems[slot]),
            pltpu.make_async_copy(term_hbm.at[sl], t_v[slot], isems[slot]),
        )

    for c in in_copies(0, wid):
        c.start()

    accs = (jnp.zeros((LANES,), jnp.float32),) * GROUPS
    for it in range(ITERS):
        chunk = wid + it * N_WORKERS
        slot = it % 2

        if it + 1 < ITERS:
            for c in in_copies(1 - slot, chunk + N_WORKERS):
                c.start()

        for c in in_copies(slot, chunk):
            c.wait()

        @plsc.parallel_loop(0, CHUNK, step=GROUPS * LANES, unroll=2,
                            carry=accs)
        def body(j, carry):
            new = []
            for g in range(GROUPS):
                sl = pl.ds(j + g * LANES, LANES)
                new.append(carry[g]
                           + f_v[slot][sl] * t_v[slot][sl].astype(jnp.float32))
            return tuple(new)

        accs = body

    a0, a1, a2, a3 = accs
    acc_v[...] = (a0 + a1) + (a2 + a3)
    pltpu.sync_copy(acc_v, part_hbm.at[wid])


@functools.partial(
    pl.kernel,
    out_type=jax.ShapeDtypeStruct((N_WORKERS, LANES), jnp.float32),
    mesh=plsc.VectorSubcoreMesh(core_axis_name="c", subcore_axis_name="s"),
    scratch_types=[
        pltpu.VMEM((CHUNK,), jnp.float32),
        pltpu.VMEM((CHUNK,), jnp.float32),
        pltpu.VMEM((CHUNK,), jnp.int32),
        pltpu.VMEM((CHUNK,), jnp.int32),
        pltpu.VMEM((LANES,), jnp.float32),
        pltpu.SemaphoreType.DMA,
        pltpu.SemaphoreType.DMA,
    ],
)
def _sc_reduce(*args):
    _sc_reduce_body(*args)


def _select_body(f_ref, m_ref, s_ref, o_ref):
    o_ref[...] = jnp.where(s_ref[...] == 0, f_ref[...], m_ref[...])


def _tc_select(fringe, minf, status):
    spec = pl.BlockSpec((TC_BLOCK,), lambda i: (i,))
    return pl.pallas_call(
        _select_body,
        grid=(TC_GRID,),
        in_specs=[spec, spec, spec],
        out_specs=spec,
        out_shape=jax.ShapeDtypeStruct((N_NODES,), jnp.float32),
    )(fringe, minf, status)


def _flux_body(step_ref, f_ref, t_ref, part_ref, flux_ref, out_ref):
    sl = pl.ds(0, TAIL_VALID)
    tail_sum = jnp.sum(f_ref[sl] * t_ref[sl].astype(jnp.float32))
    total = tail_sum + jnp.sum(part_ref[...])
    step = step_ref[0, 0]
    cols = lax.broadcasted_iota(jnp.int32, (1, 1000), 1)
    out_ref[...] = jnp.where(cols == step, total, flux_ref[...])


def _flux_update(step2d, fringe, term, partials, flux2d):
    tail_spec = pl.BlockSpec((TAIL_CHUNK,), lambda i: (TAIL_BLOCK,))
    return pl.pallas_call(
        _flux_body,
        grid=(1,),
        out_shape=jax.ShapeDtypeStruct((1, 1000), jnp.float32),
        in_specs=[
            pl.BlockSpec(memory_space=pltpu.SMEM),
            tail_spec,
            tail_spec,
            pl.BlockSpec((N_WORKERS, LANES), lambda i: (0, 0)),
            pl.BlockSpec((1, 1000), lambda i: (0, 0)),
        ],
        out_specs=pl.BlockSpec((1, 1000), lambda i: (0, 0)),
    )(step2d, fringe, term, partials, flux2d)


def kernel(fringe_thickness, min_fringe_thickness, fluxes, node_is_terminus,
           status_at_node, current_step):
    partials = _sc_reduce(fringe_thickness, node_is_terminus)
    cleared = _tc_select(fringe_thickness, min_fringe_thickness, status_at_node)
    step2d = jnp.asarray(current_step, jnp.int32).reshape(1, 1)
    flux2d = fluxes.reshape(1, 1000)
    out2d = _flux_update(step2d, fringe_thickness, node_is_terminus,
                         partials, flux2d)
    return cleared, out2d.reshape(fluxes.shape)
